# Initial kernel scaffold; baseline (speedup 1.0000x reference)
import functools

import jax
import jax.numpy as jnp
from jax import lax
from jax.experimental import pallas as pl
from jax.experimental.pallas import tpu as pltpu
from jax.experimental.pallas import tpu_sc as plsc

BB, SS, DD = 4, 4096, 2048
KCAP = 512
HID = DD // 4

_SBLK = 512


def _scores_body(x_ref, w1_ref, b1_ref, w2_ref, b2_ref, s_ref):
    xb = x_ref[...]
    h = jnp.dot(xb, w1_ref[...], precision=lax.Precision.HIGHEST,
                preferred_element_type=jnp.float32) + b1_ref[...]
    h = jnp.maximum(h, 0.0)
    s = jnp.dot(h, w2_ref[...], precision=lax.Precision.HIGHEST,
                preferred_element_type=jnp.float32)
    s_ref[...] = s + b2_ref[0]


def _scores(xf, w1, b1, w2, b2):
    n = (BB * SS) // _SBLK
    return pl.pallas_call(
        _scores_body,
        grid=(n,),
        in_specs=[
            pl.BlockSpec((_SBLK, DD), lambda i: (i, 0)),
            pl.BlockSpec((DD, HID), lambda i: (0, 0)),
            pl.BlockSpec((HID,), lambda i: (0,)),
            pl.BlockSpec((HID, 1), lambda i: (0, 0)),
            pl.BlockSpec((1,), lambda i: (0,)),
        ],
        out_specs=pl.BlockSpec((_SBLK, 1), lambda i: (i, 0)),
        out_shape=jax.ShapeDtypeStruct((BB * SS, 1), jnp.float32),
    )(xf, w1, b1, w2, b2)


def _row_cumsum(m):
    c = m
    sh = 1
    while sh < SS:
        z = jnp.zeros((BB, sh), jnp.float32)
        c = c + jnp.concatenate([z, c[:, : SS - sh]], axis=1)
        sh *= 2
    return c


def _route_body(s_ref, mask_ref, idx_ref):
    s = s_ref[...]
    u = lax.bitcast_convert_type(s, jnp.uint32)
    big = jnp.uint32(0x80000000)
    key = jnp.where(u < big, u | big, ~u)
    kf = jnp.float32(KCAP)
    t = jnp.zeros((BB, 1), jnp.uint32)
    for bit in range(31, -1, -1):
        cand = t | jnp.uint32(1 << bit)
        cnt = jnp.sum((key >= cand).astype(jnp.float32), axis=1, keepdims=True)
        t = jnp.where(cnt >= kf, cand, t)
    mask_gt = key > t
    mask_eq = key == t
    ngt = jnp.sum(mask_gt.astype(jnp.float32), axis=1, keepdims=True)
    r = kf - ngt
    cum_eq = _row_cumsum(mask_eq.astype(jnp.float32))
    mask = mask_gt | (mask_eq & (cum_eq <= r))
    mask_ref[...] = mask.astype(jnp.int32)
    pos = _row_cumsum(mask.astype(jnp.float32))
    iota_s = lax.broadcasted_iota(jnp.float32, (SS, 1), 0)
    jrow = lax.broadcasted_iota(jnp.float32, (KCAP, 1), 0) + 1.0
    for b in range(BB):
        pos_b = pos[b : b + 1, :]
        m_b = mask[b : b + 1, :]
        oh = jnp.where((pos_b == jrow) & m_b, 1.0, 0.0)
        idx_f = jnp.dot(oh, iota_s, precision=lax.Precision.HIGHEST,
                        preferred_element_type=jnp.float32)
        idx_ref[b * KCAP : (b + 1) * KCAP, :] = idx_f.astype(jnp.int32) + b * SS


def _route(scores):
    return pl.pallas_call(
        _route_body,
        out_shape=(
            jax.ShapeDtypeStruct((BB, SS), jnp.int32),
            jax.ShapeDtypeStruct((BB * KCAP, 1), jnp.int32),
        ),
    )(scores)


_MBLK, _NBLK = 512, 1024


def _layer_body(a_ref, w_ref, b_ref, o_ref):
    o_ref[...] = (
        jnp.dot(a_ref[...], w_ref[...], precision=lax.Precision.HIGHEST,
                preferred_element_type=jnp.float32)
        + b_ref[...]
    )


def _layer(a, wl, bl):
    m = BB * KCAP
    return pl.pallas_call(
        _layer_body,
        grid=(m // _MBLK, DD // _NBLK),
        in_specs=[
            pl.BlockSpec((_MBLK, DD), lambda i, j: (i, 0)),
            pl.BlockSpec((DD, _NBLK), lambda i, j: (0, j)),
            pl.BlockSpec((_NBLK,), lambda i, j: (j,)),
        ],
        out_specs=pl.BlockSpec((_MBLK, _NBLK), lambda i, j: (i, j)),
        out_shape=jax.ShapeDtypeStruct((m, DD), jnp.float32),
    )(a, wl, bl)


_NC, _NSUB = 2, 16
_NW = _NC * _NSUB
_RPW = (BB * KCAP) // _NW
_CHUNK = 32

_sc_mesh = plsc.VectorSubcoreMesh(
    core_axis_name="c", subcore_axis_name="s", num_cores=_NC, num_subcores=_NSUB
)


@functools.partial(
    pl.kernel,
    out_type=jax.ShapeDtypeStruct((BB * KCAP, DD), jnp.float32),
    mesh=_sc_mesh,
    scratch_types=[
        pltpu.VMEM((_CHUNK,), jnp.int32),
        pltpu.VMEM((_CHUNK, DD), jnp.float32),
        pltpu.SemaphoreType.DMA,
    ],
)
def _sc_gather(x_hbm, idx_hbm, out_hbm, idx_v, rows_v, sem):
    wid = lax.axis_index("s") * _NC + lax.axis_index("c")
    base = wid * _RPW
    for c in range(_RPW // _CHUNK):
        b0 = base + c * _CHUNK
        pltpu.sync_copy(idx_hbm.at[pl.ds(b0, _CHUNK)], idx_v)
        pltpu.async_copy(x_hbm.at[idx_v], rows_v, sem).wait()
        pltpu.sync_copy(rows_v, out_hbm.at[pl.ds(b0, _CHUNK)])


@functools.partial(
    pl.kernel,
    out_type=(),
    mesh=_sc_mesh,
    scratch_types=[
        pltpu.VMEM((_CHUNK,), jnp.int32),
        pltpu.VMEM((_CHUNK, DD), jnp.float32),
        pltpu.SemaphoreType.DMA,
    ],
)
def _sc_scatter(proc_hbm, idx_hbm, out_ref, idx_v, rows_v, sem):
    wid = lax.axis_index("s") * _NC + lax.axis_index("c")
    base = wid * _RPW
    for c in range(_RPW // _CHUNK):
        b0 = base + c * _CHUNK
        pltpu.sync_copy(idx_hbm.at[pl.ds(b0, _CHUNK)], idx_v)
        pltpu.sync_copy(proc_hbm.at[pl.ds(b0, _CHUNK)], rows_v)
        pltpu.async_copy(rows_v, out_ref.at[idx_v], sem).wait()


def kernel(x, w1, b1, w2, b2, wl, bl):
    xf = x.reshape(BB * SS, DD)
    scores = _scores(xf, w1, b1, w2, b2).reshape(BB, SS)
    mask_i, idx_col = _route(scores)
    idx_flat = idx_col.reshape(BB * KCAP)
    gathered = _sc_gather(xf, idx_flat)
    proc = _layer(gathered, wl, bl)
    out_ref = jax.new_ref(xf)
    _sc_scatter(proc, idx_flat, out_ref)
    out = out_ref[...].reshape(BB, SS, DD)
    return out, mask_i.astype(jnp.bool_)


# speedup vs baseline: 2.2278x; 2.2278x over previous
import functools

import jax
import jax.numpy as jnp
from jax import lax
from jax.experimental import pallas as pl
from jax.experimental.pallas import tpu as pltpu
from jax.experimental.pallas import tpu_sc as plsc

BB, SS, DD = 4, 4096, 2048
KCAP = 512
HID = DD // 4

_SBLK = 512


def _scores_body(x_ref, w1_ref, b1_ref, w2_ref, b2_ref, s_ref):
    xb = x_ref[...]
    hT = lax.dot_general(w1_ref[...], xb, (((0,), (1,)), ((), ())),
                         precision=lax.Precision.DEFAULT,
                         preferred_element_type=jnp.float32)
    hT = jnp.maximum(hT + b1_ref[...].reshape(HID, 1), 0.0)
    sT = lax.dot_general(w2_ref[...], hT, (((0,), (0,)), ((), ())),
                         precision=lax.Precision.DEFAULT,
                         preferred_element_type=jnp.float32)
    s_ref[...] = sT + b2_ref[0]


def _scores(xf, w1, b1, w2, b2):
    n = (BB * SS) // _SBLK
    return pl.pallas_call(
        _scores_body,
        grid=(n,),
        in_specs=[
            pl.BlockSpec((_SBLK, DD), lambda i: (i, 0)),
            pl.BlockSpec((DD, HID), lambda i: (0, 0)),
            pl.BlockSpec((HID,), lambda i: (0,)),
            pl.BlockSpec((HID, 1), lambda i: (0, 0)),
            pl.BlockSpec((1,), lambda i: (0,)),
        ],
        out_specs=pl.BlockSpec((1, _SBLK), lambda i: (0, i)),
        out_shape=jax.ShapeDtypeStruct((1, BB * SS), jnp.float32),
    )(xf, w1, b1, w2, b2)


def _row_cumsum(m):
    c = m
    sh = 1
    while sh < SS:
        z = jnp.zeros((BB, sh), jnp.float32)
        c = c + jnp.concatenate([z, c[:, : SS - sh]], axis=1)
        sh *= 2
    return c


def _route_body(s_ref, mask_ref, idx_ref):
    s = s_ref[...]
    u = lax.bitcast_convert_type(s, jnp.uint32)
    big = jnp.uint32(0x80000000)
    key = jnp.where(u < big, u | big, ~u)
    kf = jnp.float32(KCAP)
    t = jnp.zeros((BB, 1), jnp.uint32)
    for bit in range(31, -1, -1):
        cand = t | jnp.uint32(1 << bit)
        cnt = jnp.sum((key >= cand).astype(jnp.float32), axis=1, keepdims=True)
        t = jnp.where(cnt >= kf, cand, t)
    mask_gt = key > t
    mask_eq = key == t
    ngt = jnp.sum(mask_gt.astype(jnp.float32), axis=1, keepdims=True)
    r = kf - ngt
    cum_eq = _row_cumsum(mask_eq.astype(jnp.float32))
    mask = mask_gt | (mask_eq & (cum_eq <= r))
    mask_ref[...] = mask.astype(jnp.int32)
    pos = _row_cumsum(mask.astype(jnp.float32))
    iota_s = lax.broadcasted_iota(jnp.int32, (SS, 1), 0).astype(jnp.float32)
    jrow = lax.broadcasted_iota(jnp.int32, (KCAP, 1), 0).astype(jnp.float32) + 1.0
    for b in range(BB):
        pos_b = pos[b : b + 1, :]
        m_b = mask[b : b + 1, :]
        oh = jnp.where((pos_b == jrow) & m_b, 1.0, 0.0)
        idx_f = jnp.dot(oh, iota_s, precision=lax.Precision.HIGHEST,
                        preferred_element_type=jnp.float32)
        idx_ref[b * KCAP : (b + 1) * KCAP, :] = idx_f.astype(jnp.int32) + b * SS


def _route(scores):
    return pl.pallas_call(
        _route_body,
        out_shape=(
            jax.ShapeDtypeStruct((BB, SS), jnp.int32),
            jax.ShapeDtypeStruct((BB * KCAP, 1), jnp.int32),
        ),
    )(scores)


_MBLK, _NBLK = 512, 1024


def _layer_body(a_ref, w_ref, b_ref, o_ref):
    o_ref[...] = (
        jnp.dot(a_ref[...], w_ref[...], precision=lax.Precision.DEFAULT,
                preferred_element_type=jnp.float32)
        + b_ref[...]
    )


def _layer(a, wl, bl):
    m = BB * KCAP
    return pl.pallas_call(
        _layer_body,
        grid=(m // _MBLK, DD // _NBLK),
        in_specs=[
            pl.BlockSpec((_MBLK, DD), lambda i, j: (i, 0)),
            pl.BlockSpec((DD, _NBLK), lambda i, j: (0, j)),
            pl.BlockSpec((_NBLK,), lambda i, j: (j,)),
        ],
        out_specs=pl.BlockSpec((_MBLK, _NBLK), lambda i, j: (i, j)),
        out_shape=jax.ShapeDtypeStruct((m, DD), jnp.float32),
    )(a, wl, bl)


_NC, _NSUB = 2, 16
_NW = _NC * _NSUB
_RPW = (BB * KCAP) // _NW
_CHUNK = 32

@functools.lru_cache(maxsize=None)
def _sc_kernels():
    mesh = plsc.VectorSubcoreMesh(
        core_axis_name="c", subcore_axis_name="s",
        num_cores=_NC, num_subcores=_NSUB,
    )
    scratch = [
        pltpu.VMEM((_CHUNK,), jnp.int32),
        pltpu.VMEM((_CHUNK, DD), jnp.float32),
        pltpu.SemaphoreType.DMA,
    ]

    @functools.partial(
        pl.kernel,
        out_type=jax.ShapeDtypeStruct((BB * KCAP, DD), jnp.float32),
        mesh=mesh,
        scratch_types=scratch,
    )
    def sc_gather(x_hbm, idx_hbm, out_hbm, idx_v, rows_v, sem):
        wid = lax.axis_index("s") * _NC + lax.axis_index("c")
        base = wid * _RPW
        for c in range(_RPW // _CHUNK):
            b0 = base + c * _CHUNK
            pltpu.sync_copy(idx_hbm.at[pl.ds(b0, _CHUNK)], idx_v)
            pltpu.async_copy(x_hbm.at[idx_v], rows_v, sem).wait()
            pltpu.sync_copy(rows_v, out_hbm.at[pl.ds(b0, _CHUNK)])

    @functools.partial(
        pl.kernel,
        out_type=(),
        mesh=mesh,
        scratch_types=scratch,
    )
    def sc_scatter(proc_hbm, idx_hbm, out_ref, idx_v, rows_v, sem):
        wid = lax.axis_index("s") * _NC + lax.axis_index("c")
        base = wid * _RPW
        for c in range(_RPW // _CHUNK):
            b0 = base + c * _CHUNK
            pltpu.sync_copy(idx_hbm.at[pl.ds(b0, _CHUNK)], idx_v)
            pltpu.sync_copy(proc_hbm.at[pl.ds(b0, _CHUNK)], rows_v)
            pltpu.async_copy(rows_v, out_ref.at[idx_v], sem).wait()

    return sc_gather, sc_scatter


def _sc_gather(x, idx):
    return _sc_kernels()[0](x, idx)


def _sc_scatter(proc, idx, out_ref):
    return _sc_kernels()[1](proc, idx, out_ref)


def kernel(x, w1, b1, w2, b2, wl, bl):
    xf = x.reshape(BB * SS, DD)
    scores = _scores(xf, w1, b1, w2, b2).reshape(BB, SS)

    mask_i, idx_col = _route(scores)
    idx_flat = idx_col.reshape(BB * KCAP)
    gathered = _sc_gather(xf, idx_flat)
    proc = _layer(gathered, wl, bl)
    out_ref = jax.new_ref(xf)
    _sc_scatter(proc, idx_flat, out_ref)
    out = out_ref[...].reshape(BB, SS, DD)
    return out, mask_i.astype(jnp.bool_)


# speedup vs baseline: 3.1729x; 1.4242x over previous
import functools

import jax
import jax.numpy as jnp
from jax import lax
from jax.experimental import pallas as pl
from jax.experimental.pallas import tpu as pltpu
from jax.experimental.pallas import tpu_sc as plsc

BB, SS, DD = 4, 4096, 2048
KCAP = 512
HID = DD // 4

_SBLK = 512


def _scores_body(x_ref, w1_ref, b1_ref, w2_ref, b2_ref, s_ref, xc_ref):
    xb = x_ref[...]
    hT = lax.dot_general(w1_ref[...], xb, (((0,), (1,)), ((), ())),
                         precision=lax.Precision.DEFAULT,
                         preferred_element_type=jnp.float32)
    hT = jnp.maximum(hT + b1_ref[...].reshape(HID, 1), 0.0)
    sT = lax.dot_general(w2_ref[...], hT, (((0,), (0,)), ((), ())),
                         precision=lax.Precision.DEFAULT,
                         preferred_element_type=jnp.float32)
    s_ref[...] = sT + b2_ref[0]
    xc_ref[...] = xb


def _scores(xf, w1, b1, w2, b2):
    n = (BB * SS) // _SBLK
    return pl.pallas_call(
        _scores_body,
        grid=(n,),
        in_specs=[
            pl.BlockSpec((_SBLK, DD), lambda i: (i, 0)),
            pl.BlockSpec((DD, HID), lambda i: (0, 0)),
            pl.BlockSpec((HID,), lambda i: (0,)),
            pl.BlockSpec((HID, 1), lambda i: (0, 0)),
            pl.BlockSpec((1,), lambda i: (0,)),
        ],
        out_specs=(pl.BlockSpec((1, _SBLK), lambda i: (0, i)),
                   pl.BlockSpec((_SBLK, DD), lambda i: (i, 0))),
        out_shape=(jax.ShapeDtypeStruct((1, BB * SS), jnp.float32),
                   jax.ShapeDtypeStruct((BB * SS, DD), jnp.float32)),
    )(xf, w1, b1, w2, b2)


def _row_cumsum(m):
    c = m
    sh = 1
    while sh < SS:
        z = jnp.zeros((BB, sh), jnp.float32)
        c = c + jnp.concatenate([z, c[:, : SS - sh]], axis=1)
        sh *= 2
    return c


def _route_body(s_ref, mask_ref, idx_ref):
    s = s_ref[...]
    u = lax.bitcast_convert_type(s, jnp.uint32)
    big = jnp.uint32(0x80000000)
    key = jnp.where(u < big, u | big, ~u)
    kf = jnp.float32(KCAP)
    t = jnp.zeros((BB, 1), jnp.uint32)
    for bit in range(31, -1, -1):
        cand = t | jnp.uint32(1 << bit)
        cnt = jnp.sum((key >= cand).astype(jnp.float32), axis=1, keepdims=True)
        t = jnp.where(cnt >= kf, cand, t)
    mask_gt = key > t
    mask_eq = key == t
    ngt = jnp.sum(mask_gt.astype(jnp.float32), axis=1, keepdims=True)
    r = kf - ngt
    cum_eq = _row_cumsum(mask_eq.astype(jnp.float32))
    mask = mask_gt | (mask_eq & (cum_eq <= r))
    mask_ref[...] = mask.astype(jnp.int32)
    pos = _row_cumsum(mask.astype(jnp.float32))
    iota_r = lax.broadcasted_iota(jnp.int32, (1, SS), 1).astype(jnp.float32)
    jrow = lax.broadcasted_iota(jnp.int32, (KCAP, 1), 0).astype(jnp.float32) + 1.0
    for b in range(BB):
        pos_b = pos[b : b + 1, :]
        m_b = mask[b : b + 1, :]
        sel = jnp.where((pos_b == jrow) & m_b, iota_r, 0.0)
        idx_f = jnp.sum(sel, axis=1, keepdims=True)
        idx_ref[b * KCAP : (b + 1) * KCAP, :] = idx_f.astype(jnp.int32) + b * SS


def _route(scores):
    return pl.pallas_call(
        _route_body,
        out_shape=(
            jax.ShapeDtypeStruct((BB, SS), jnp.int32),
            jax.ShapeDtypeStruct((BB * KCAP, 1), jnp.int32),
        ),
    )(scores)


_MBLK, _NBLK = 512, 1024


def _layer_body(a_ref, w_ref, b_ref, o_ref):
    o_ref[...] = (
        jnp.dot(a_ref[...], w_ref[...], precision=lax.Precision.DEFAULT,
                preferred_element_type=jnp.float32)
        + b_ref[...]
    )


def _layer(a, wl, bl):
    m = BB * KCAP
    return pl.pallas_call(
        _layer_body,
        grid=(m // _MBLK, DD // _NBLK),
        in_specs=[
            pl.BlockSpec((_MBLK, DD), lambda i, j: (i, 0)),
            pl.BlockSpec((DD, _NBLK), lambda i, j: (0, j)),
            pl.BlockSpec((_NBLK,), lambda i, j: (j,)),
        ],
        out_specs=pl.BlockSpec((_MBLK, _NBLK), lambda i, j: (i, j)),
        out_shape=jax.ShapeDtypeStruct((m, DD), jnp.float32),
    )(a, wl, bl)


_NC, _NSUB = 2, 16
_NW = _NC * _NSUB
_RPW = (BB * KCAP) // _NW
_CHUNK = 32

@functools.lru_cache(maxsize=None)
def _sc_kernels():
    mesh = plsc.VectorSubcoreMesh(
        core_axis_name="c", subcore_axis_name="s",
        num_cores=_NC, num_subcores=_NSUB,
    )
    scratch = [
        pltpu.VMEM((_CHUNK,), jnp.int32),
        pltpu.VMEM((_CHUNK, DD), jnp.float32),
        pltpu.SemaphoreType.DMA,
    ]

    @functools.partial(
        pl.kernel,
        out_type=jax.ShapeDtypeStruct((BB * KCAP, DD), jnp.float32),
        mesh=mesh,
        scratch_types=scratch,
    )
    def sc_gather(x_hbm, idx_hbm, out_hbm, idx_v, rows_v, sem):
        wid = lax.axis_index("s") * _NC + lax.axis_index("c")
        base = wid * _RPW
        for c in range(_RPW // _CHUNK):
            b0 = base + c * _CHUNK
            pltpu.sync_copy(idx_hbm.at[pl.ds(b0, _CHUNK)], idx_v)
            pltpu.async_copy(x_hbm.at[idx_v], rows_v, sem).wait()
            pltpu.sync_copy(rows_v, out_hbm.at[pl.ds(b0, _CHUNK)])

    @functools.partial(
        pl.kernel,
        out_type=(),
        mesh=mesh,
        scratch_types=scratch,
    )
    def sc_scatter(proc_hbm, idx_hbm, out_ref, idx_v, rows_v, sem):
        wid = lax.axis_index("s") * _NC + lax.axis_index("c")
        base = wid * _RPW
        for c in range(_RPW // _CHUNK):
            b0 = base + c * _CHUNK
            pltpu.sync_copy(idx_hbm.at[pl.ds(b0, _CHUNK)], idx_v)
            pltpu.sync_copy(proc_hbm.at[pl.ds(b0, _CHUNK)], rows_v)
            pltpu.async_copy(rows_v, out_ref.at[idx_v], sem).wait()

    return sc_gather, sc_scatter


def _sc_gather(x, idx):
    return _sc_kernels()[0](x, idx)


def _sc_scatter(proc, idx, out_ref):
    return _sc_kernels()[1](proc, idx, out_ref)


def kernel(x, w1, b1, w2, b2, wl, bl):
    xf = x.reshape(BB * SS, DD)
    scores_row, xcopy = _scores(xf, w1, b1, w2, b2)
    scores = scores_row.reshape(BB, SS)
    mask_i, idx_col = _route(scores)
    idx_flat = idx_col.reshape(BB * KCAP)
    gathered = _sc_gather(xf, idx_flat)
    proc = _layer(gathered, wl, bl)
    out_ref = jax.new_ref(xcopy)
    _sc_scatter(proc, idx_flat, out_ref)
    out = out_ref[...].reshape(BB, SS, DD)
    return out, mask_i.astype(jnp.bool_)
